# baseline (speedup 1.0000x reference)
import functools

import jax
import jax.numpy as jnp
from jax import lax
from jax.experimental import pallas as pl
from jax.experimental.pallas import tpu as pltpu
from jax.experimental.pallas import tpu_sc as plsc

_VOCAB = 1000000
_DIM = 64
_B = 16384
_NC, _NS = 2, 16
_NW = _NC * _NS
_BPW = _B // _NW
_CH = 128
_NCHUNK = _BPW // _CH
_LANES = 16


def _sc_partials(center2d, pos2d, neg2d, input_emb, output_emb):
    mesh = plsc.VectorSubcoreMesh(core_axis_name="c", subcore_axis_name="s")

    @functools.partial(
        pl.kernel,
        out_type=[
            jax.ShapeDtypeStruct((_B, _LANES), jnp.float32),
            jax.ShapeDtypeStruct((_B, _LANES), jnp.float32),
        ],
        mesh=mesh,
        compiler_params=pltpu.CompilerParams(use_tc_tiling_on_sc=False),
        scratch_types=[
            pltpu.VMEM((_NCHUNK, _CH), jnp.int32),
            pltpu.VMEM((_NCHUNK, _CH), jnp.int32),
            pltpu.VMEM((_NCHUNK, _CH), jnp.int32),
            pltpu.VMEM((_BPW, _DIM), jnp.float32),
            pltpu.VMEM((_BPW, _DIM), jnp.float32),
            pltpu.VMEM((_BPW, _DIM), jnp.float32),
            pltpu.VMEM((_BPW, _LANES), jnp.float32),
            pltpu.VMEM((_BPW, _LANES), jnp.float32),
            pltpu.SemaphoreType.DMA,
        ],
    )
    def k(center_hbm, pos_hbm, neg_hbm, in_emb, out_emb, opos_hbm, oneg_hbm,
          cidx, pidx, nidx, vrows, prows, nrows, tpos, tneg, sem):
        wid = lax.axis_index("s") * _NC + lax.axis_index("c")
        base = wid * _BPW

        for j in range(_NCHUNK):
            row = wid * _NCHUNK + j
            pltpu.sync_copy(center_hbm.at[row], cidx.at[j])
            pltpu.sync_copy(pos_hbm.at[row], pidx.at[j])
            pltpu.sync_copy(neg_hbm.at[row], nidx.at[j])

        copies = []
        for j in range(_NCHUNK):
            sl = pl.ds(j * _CH, _CH)
            copies.append(pltpu.async_copy(in_emb.at[cidx.at[j]], vrows.at[sl], sem))
            copies.append(pltpu.async_copy(out_emb.at[pidx.at[j]], prows.at[sl], sem))
            copies.append(pltpu.async_copy(out_emb.at[nidx.at[j]], nrows.at[sl], sem))
        for c in copies:
            c.wait()

        def body(i, carry):
            v = [vrows[i, pl.ds(m * _LANES, _LANES)] for m in range(4)]
            p = [prows[i, pl.ds(m * _LANES, _LANES)] for m in range(4)]
            n = [nrows[i, pl.ds(m * _LANES, _LANES)] for m in range(4)]
            ap = v[0] * p[0] + v[1] * p[1] + v[2] * p[2] + v[3] * p[3]
            an = v[0] * n[0] + v[1] * n[1] + v[2] * n[2] + v[3] * n[3]
            tpos[i, :] = ap
            tneg[i, :] = an
            return carry

        lax.fori_loop(0, _BPW, body, 0)

        pltpu.sync_copy(tpos, opos_hbm.at[pl.ds(base, _BPW)])
        pltpu.sync_copy(tneg, oneg_hbm.at[pl.ds(base, _BPW)])

    return k(center2d, pos2d, neg2d, input_emb, output_emb)


def _tc_loss(tpos, tneg):
    def body(p_ref, n_ref, o_ref):
        ps = jnp.sum(p_ref[...], axis=1)
        ns = jnp.sum(n_ref[...], axis=1)
        sp = jax.nn.sigmoid(ps)
        sn = jax.nn.sigmoid(ns)
        loss = (-jnp.mean(jnp.log(sp + 1e-09))
                - jnp.mean(jnp.log(1.0 - sn + 1e-09)))
        o_ref[...] = jnp.broadcast_to(loss, (1, 1))

    out = pl.pallas_call(
        body,
        out_shape=jax.ShapeDtypeStruct((1, 1), jnp.float32),
    )(tpos, tneg)
    return out[0, 0]


def kernel(center, pos, neg, input_emb, output_emb):
    center2d = center.astype(jnp.int32).reshape(_NW * _NCHUNK, _CH)
    pos2d = pos.astype(jnp.int32).reshape(_NW * _NCHUNK, _CH)
    neg2d = neg.astype(jnp.int32).reshape(_NW * _NCHUNK, _CH)
    tpos, tneg = _sc_partials(center2d, pos2d, neg2d, input_emb, output_emb)
    return _tc_loss(tpos, tneg)


# speedup vs baseline: 1.5774x; 1.5774x over previous
import functools

import jax
import jax.numpy as jnp
from jax import lax
from jax.experimental import pallas as pl
from jax.experimental.pallas import tpu as pltpu
from jax.experimental.pallas import tpu_sc as plsc

_VOCAB = 1000000
_DIM = 64
_B = 16384
_NC, _NS = 2, 16
_NW = _NC * _NS
_BPW = _B // _NW
_CH = 128
_NCHUNK = _BPW // _CH
_LANES = 16
_OROW = _BPW * _LANES // 128


def _sc_partials(center2d, pos2d, neg2d, input_emb, output_emb):
    mesh = plsc.VectorSubcoreMesh(core_axis_name="c", subcore_axis_name="s")

    @functools.partial(
        pl.kernel,
        out_type=[
            jax.ShapeDtypeStruct((_B * _LANES // 128, 128), jnp.float32),
            jax.ShapeDtypeStruct((_B * _LANES // 128, 128), jnp.float32),
        ],
        mesh=mesh,
        compiler_params=pltpu.CompilerParams(needs_layout_passes=False),
        scratch_types=[
            pltpu.VMEM((2 * _CH,), jnp.int32),
            pltpu.VMEM((2 * _CH,), jnp.int32),
            pltpu.VMEM((2 * _CH,), jnp.int32),
            pltpu.VMEM((2 * _CH, _DIM), jnp.float32),
            pltpu.VMEM((2 * _CH, _DIM), jnp.float32),
            pltpu.VMEM((2 * _CH, _DIM), jnp.float32),
            pltpu.VMEM((_OROW, 128), jnp.float32),
            pltpu.VMEM((_OROW, 128), jnp.float32),
            pltpu.SemaphoreType.DMA,
            pltpu.SemaphoreType.DMA,
        ],
    )
    def k(center_hbm, pos_hbm, neg_hbm, in_emb, out_emb, dummy_hbm,
          opos_hbm, oneg_hbm,
          cids, pids, nids, vbuf, pbuf, nbuf, tpos, tneg, sem0, sem1):
        wid = lax.axis_index("s") * _NC + lax.axis_index("c")
        sems = [sem0, sem1]

        def stage(j, slot):
            row = wid * _NCHUNK + j
            sl = pl.ds(slot * _CH, _CH)
            pltpu.sync_copy(center_hbm.at[row], cids.at[sl])
            pltpu.sync_copy(pos_hbm.at[row], pids.at[sl])
            pltpu.sync_copy(neg_hbm.at[row], nids.at[sl])

        def scalar_at(ref, i):
            lane = jnp.full((_LANES,), i, jnp.int32)
            return jnp.max(plsc.load_gather(ref, [lane]))

        def fire(slot):
            def fbody(i, carry):
                r = slot * _CH + i
                ci = scalar_at(cids, r)
                pi = scalar_at(pids, r)
                ni = scalar_at(nids, r)
                pltpu.async_copy(in_emb.at[ci], vbuf.at[r], sems[slot])
                pltpu.async_copy(out_emb.at[pi], pbuf.at[r], sems[slot])
                pltpu.async_copy(out_emb.at[ni], nbuf.at[r], sems[slot])
                return carry

            lax.fori_loop(0, _CH, fbody, 0)

        def drain(slot):
            def dbody(i, carry):
                r = slot * _CH + i
                pltpu.make_async_copy(dummy_hbm, vbuf.at[r], sems[slot]).wait()
                pltpu.make_async_copy(dummy_hbm, pbuf.at[r], sems[slot]).wait()
                pltpu.make_async_copy(dummy_hbm, nbuf.at[r], sems[slot]).wait()
                return carry

            lax.fori_loop(0, _CH, dbody, 0)

        def compute(j, slot):
            def body(i, carry):
                r = slot * _CH + i
                v = [vbuf[r, pl.ds(m * _LANES, _LANES)] for m in range(4)]
                p = [pbuf[r, pl.ds(m * _LANES, _LANES)] for m in range(4)]
                n = [nbuf[r, pl.ds(m * _LANES, _LANES)] for m in range(4)]
                ap = v[0] * p[0] + v[1] * p[1] + v[2] * p[2] + v[3] * p[3]
                an = v[0] * n[0] + v[1] * n[1] + v[2] * n[2] + v[3] * n[3]
                flat = (j * _CH + i) * _LANES
                r = flat // 128
                c = flat % 128
                tpos[r, pl.ds(c, _LANES)] = ap
                tneg[r, pl.ds(c, _LANES)] = an
                return carry

            lax.fori_loop(0, _CH, body, 0)

        stage(0, 0)
        fire(0)
        for j in range(_NCHUNK):
            slot = j % 2
            if j + 1 < _NCHUNK:
                stage(j + 1, (j + 1) % 2)
                fire((j + 1) % 2)
            drain(slot)
            compute(j, slot)

        base = wid * _OROW
        pltpu.sync_copy(tpos, opos_hbm.at[pl.ds(base, _OROW)])
        pltpu.sync_copy(tneg, oneg_hbm.at[pl.ds(base, _OROW)])

    dummy = jnp.zeros((_DIM,), jnp.float32)
    return k(center2d, pos2d, neg2d, input_emb, output_emb, dummy)


def _tc_loss(tpos, tneg):
    def body(p_ref, n_ref, o_ref):
        gk = jax.lax.broadcasted_iota(jnp.int32, (128, 8), 0) // _LANES
        gg = jax.lax.broadcasted_iota(jnp.int32, (128, 8), 1)
        g = (gk == gg).astype(jnp.float32)
        ps = jnp.dot(p_ref[...], g, preferred_element_type=jnp.float32)
        ns = jnp.dot(n_ref[...], g, preferred_element_type=jnp.float32)
        sp = jax.nn.sigmoid(ps)
        sn = jax.nn.sigmoid(ns)
        loss = (-jnp.mean(jnp.log(sp + 1e-09))
                - jnp.mean(jnp.log(1.0 - sn + 1e-09)))
        o_ref[...] = jnp.broadcast_to(loss, (1, 1))

    out = pl.pallas_call(
        body,
        out_shape=jax.ShapeDtypeStruct((1, 1), jnp.float32),
    )(tpos, tneg)
    return out[0, 0]


def kernel(center, pos, neg, input_emb, output_emb):
    center2d = center.astype(jnp.int32).reshape(_NW * _NCHUNK, _CH)
    pos2d = pos.astype(jnp.int32).reshape(_NW * _NCHUNK, _CH)
    neg2d = neg.astype(jnp.int32).reshape(_NW * _NCHUNK, _CH)
    tpos, tneg = _sc_partials(center2d, pos2d, neg2d, input_emb, output_emb)
    return _tc_loss(tpos, tneg)
